# baseline (speedup 1.0000x reference)
import jax
import jax.numpy as jnp
from jax import lax
from jax.experimental import pallas as pl
from jax.experimental.pallas import tpu as pltpu

B, D, V = 64, 32, 8192
TEMP = 2.0
BB = 8
UNROLL = 16
NREG = D // UNROLL
NRING = 2


def _step_kernel(x_ref, th_ref, au_ref, gu_hbm, out_ref, *scratch):
    bufs, sems = scratch[:NRING], scratch[NRING:]
    i = pl.program_id(0)
    th = th_ref[...]

    def gu_dma(r):
        return pltpu.make_async_copy(
            gu_hbm.at[pl.ds(i * BB, BB), pl.ds(r * UNROLL * V, UNROLL * V)],
            bufs[r % NRING],
            sems[r % NRING],
        )

    for r in range(NRING - 1):
        gu_dma(r).start()

    iota_v2 = lax.broadcasted_iota(jnp.int32, (D, V), 1)
    c_cols, curv_cols = [], []
    for bb in range(BB):
        xb = x_ref[bb]
        c_cols.append(jnp.sum(xb * th, axis=1, keepdims=True))
        curv_cols.append(jnp.min(jnp.where(xb > 0.5, iota_v2, V),
                                 axis=1, keepdims=True))
    c2 = jnp.concatenate(c_cols, axis=1).T
    curv2 = jnp.concatenate(curv_cols, axis=1).T

    E = jnp.exp(th * (1.0 / TEMP))
    SEt = jnp.sum(E, axis=1, keepdims=True).T
    s2 = jnp.exp(c2 * (-1.0 / TEMP))

    iota_l = lax.broadcasted_iota(jnp.int32, (BB, V), 1)
    stats = []
    for r in range(NREG):
        if r + NRING - 1 < NREG:
            gu_dma(r + NRING - 1).start()
        gu_dma(r).wait()
        base = r * UNROLL
        for d in range(base, base + UNROLL):
            k = d - base
            gu_d = bufs[r % NRING][:, k * V:(k + 1) * V]
            th_row = jnp.broadcast_to(th[d:d + 1, :], (BB, V))
            E_row = jnp.broadcast_to(E[d:d + 1, :], (BB, V))
            s_d = lax.slice(s2, (0, d), (BB, d + 1))
            cv_d = lax.slice(curv2, (0, d), (BB, d + 1))
            num = jnp.where(iota_l == cv_d, 0.0, E_row) * s_d
            sc = num / (-jnp.log(gu_d))
            cmax = jnp.max(sc, axis=1, keepdims=True)
            carg = jnp.min(jnp.where(sc == cmax, iota_l, V),
                           axis=1, keepdims=True)
            th_at = jnp.sum(jnp.where(iota_l == carg, th_row, 0.0),
                            axis=1, keepdims=True)
            stats.append((cmax, carg, th_at))

    gmax, vstar, th_new = stats[0]
    dstar = jnp.zeros((BB, 1), jnp.int32)
    for d in range(1, D):
        cmax, carg, th_at = stats[d]
        upd = cmax > gmax
        gmax = jnp.where(upd, cmax, gmax)
        dstar = jnp.where(upd, d, dstar)
        vstar = jnp.where(upd, carg, vstar)
        th_new = jnp.where(upd, th_at, th_new)

    iota_d1 = lax.broadcasted_iota(jnp.int32, (BB, D), 1)
    at_d = iota_d1 == dstar
    zmat = s2 * SEt - 1.0
    zsum = jnp.sum(zmat, axis=1, keepdims=True)
    rowz_at = jnp.sum(jnp.where(at_d, zmat, 0.0), axis=1, keepdims=True)
    th_old = jnp.sum(jnp.where(at_d, c2, 0.0), axis=1, keepdims=True)

    lse_f = jnp.log(zsum)
    delta = (th_old - th_new) / TEMP
    lp_forward = -delta - lse_f
    z2 = zsum - rowz_at + ((rowz_at + 1.0) * jnp.exp(delta) - 1.0)
    lp_reverse = delta - jnp.log(z2)
    la = (th_new - th_old) + lp_reverse - lp_forward
    accept = jnp.exp(la) > au_ref[...]

    iota_d2 = lax.broadcasted_iota(jnp.int32, (D, 1), 0)
    for bb in range(BB):
        acc_b = lax.slice(accept, (bb, 0), (bb + 1, 1))
        ds_b = lax.slice(dstar, (bb, 0), (bb + 1, 1))
        vs_b = lax.slice(vstar, (bb, 0), (bb + 1, 1))
        flip = (iota_d2 == ds_b) & acc_b
        row_idx = jnp.where(flip, vs_b, curv_cols[bb])
        out_ref[bb] = (iota_v2 == row_idx).astype(jnp.float32)


@jax.jit
def kernel(x, theta, gumbel_u, accept_u):
    th = theta.reshape(D, V)
    au = accept_u.reshape(B, 1)
    return pl.pallas_call(
        _step_kernel,
        grid=(B // BB,),
        in_specs=[
            pl.BlockSpec((BB, D, V), lambda i: (i, 0, 0)),
            pl.BlockSpec((D, V), lambda i: (0, 0)),
            pl.BlockSpec((BB, 1), lambda i: (i, 0)),
            pl.BlockSpec(memory_space=pl.ANY),
        ],
        out_specs=pl.BlockSpec((BB, D, V), lambda i: (i, 0, 0)),
        out_shape=jax.ShapeDtypeStruct((B, D, V), x.dtype),
        scratch_shapes=(
            [pltpu.VMEM((BB, UNROLL * V), jnp.float32)] * NRING
            + [pltpu.SemaphoreType.DMA] * NRING
        ),
    )(x, th, au, gumbel_u)


# speedup vs baseline: 1.0311x; 1.0311x over previous
import jax
import jax.numpy as jnp
from jax import lax
from jax.experimental import pallas as pl
from jax.experimental.pallas import tpu as pltpu

B, D, V = 64, 32, 8192
TEMP = 2.0
BB = 8
UNROLL = 8
NREG = D // UNROLL
NRING = 3


def _step_kernel(x_ref, th_ref, au_ref, gu_hbm, out_ref, *scratch):
    bufs, sems = scratch[:NRING], scratch[NRING:]
    i = pl.program_id(0)
    th = th_ref[...]

    def gu_dma(r):
        return pltpu.make_async_copy(
            gu_hbm.at[pl.ds(i * BB, BB), pl.ds(r * UNROLL * V, UNROLL * V)],
            bufs[r % NRING],
            sems[r % NRING],
        )

    for r in range(NRING - 1):
        gu_dma(r).start()

    iota_v2 = lax.broadcasted_iota(jnp.int32, (D, V), 1)
    c_cols, curv_cols = [], []
    for bb in range(BB):
        xb = x_ref[bb]
        c_cols.append(jnp.sum(xb * th, axis=1, keepdims=True))
        curv_cols.append(jnp.min(jnp.where(xb > 0.5, iota_v2, V),
                                 axis=1, keepdims=True))
    c2 = jnp.concatenate(c_cols, axis=1).T
    curv2 = jnp.concatenate(curv_cols, axis=1).T

    E = jnp.exp(th * (1.0 / TEMP))
    SEt = jnp.sum(E, axis=1, keepdims=True).T
    s2 = jnp.exp(c2 * (-1.0 / TEMP))

    iota_l = lax.broadcasted_iota(jnp.int32, (BB, V), 1)
    stats = []
    for r in range(NREG):
        if r + NRING - 1 < NREG:
            gu_dma(r + NRING - 1).start()
        gu_dma(r).wait()
        base = r * UNROLL
        for d in range(base, base + UNROLL):
            k = d - base
            gu_d = bufs[r % NRING][:, k * V:(k + 1) * V]
            th_row = jnp.broadcast_to(th[d:d + 1, :], (BB, V))
            E_row = jnp.broadcast_to(E[d:d + 1, :], (BB, V))
            s_d = lax.slice(s2, (0, d), (BB, d + 1))
            cv_d = lax.slice(curv2, (0, d), (BB, d + 1))
            num = jnp.where(iota_l == cv_d, 0.0, E_row) * s_d
            sc = num / (-jnp.log(gu_d))
            cmax = jnp.max(sc, axis=1, keepdims=True)
            carg = jnp.min(jnp.where(sc == cmax, iota_l, V),
                           axis=1, keepdims=True)
            th_at = jnp.sum(jnp.where(iota_l == carg, th_row, 0.0),
                            axis=1, keepdims=True)
            stats.append((cmax, carg, th_at))

    gmax, vstar, th_new = stats[0]
    dstar = jnp.zeros((BB, 1), jnp.int32)
    for d in range(1, D):
        cmax, carg, th_at = stats[d]
        upd = cmax > gmax
        gmax = jnp.where(upd, cmax, gmax)
        dstar = jnp.where(upd, d, dstar)
        vstar = jnp.where(upd, carg, vstar)
        th_new = jnp.where(upd, th_at, th_new)

    iota_d1 = lax.broadcasted_iota(jnp.int32, (BB, D), 1)
    at_d = iota_d1 == dstar
    zmat = s2 * SEt - 1.0
    zsum = jnp.sum(zmat, axis=1, keepdims=True)
    rowz_at = jnp.sum(jnp.where(at_d, zmat, 0.0), axis=1, keepdims=True)
    th_old = jnp.sum(jnp.where(at_d, c2, 0.0), axis=1, keepdims=True)

    lse_f = jnp.log(zsum)
    delta = (th_old - th_new) / TEMP
    lp_forward = -delta - lse_f
    z2 = zsum - rowz_at + ((rowz_at + 1.0) * jnp.exp(delta) - 1.0)
    lp_reverse = delta - jnp.log(z2)
    la = (th_new - th_old) + lp_reverse - lp_forward
    accept = jnp.exp(la) > au_ref[...]

    iota_d2 = lax.broadcasted_iota(jnp.int32, (D, 1), 0)
    for bb in range(BB):
        acc_b = lax.slice(accept, (bb, 0), (bb + 1, 1))
        ds_b = lax.slice(dstar, (bb, 0), (bb + 1, 1))
        vs_b = lax.slice(vstar, (bb, 0), (bb + 1, 1))
        flip = (iota_d2 == ds_b) & acc_b
        row_idx = jnp.where(flip, vs_b, curv_cols[bb])
        out_ref[bb] = (iota_v2 == row_idx).astype(jnp.float32)


@jax.jit
def kernel(x, theta, gumbel_u, accept_u):
    th = theta.reshape(D, V)
    au = accept_u.reshape(B, 1)
    return pl.pallas_call(
        _step_kernel,
        grid=(B // BB,),
        in_specs=[
            pl.BlockSpec((BB, D, V), lambda i: (i, 0, 0)),
            pl.BlockSpec((D, V), lambda i: (0, 0)),
            pl.BlockSpec((BB, 1), lambda i: (i, 0)),
            pl.BlockSpec(memory_space=pl.ANY),
        ],
        out_specs=pl.BlockSpec((BB, D, V), lambda i: (i, 0, 0)),
        out_shape=jax.ShapeDtypeStruct((B, D, V), x.dtype),
        scratch_shapes=(
            [pltpu.VMEM((BB, UNROLL * V), jnp.float32)] * NRING
            + [pltpu.SemaphoreType.DMA] * NRING
        ),
    )(x, th, au, gumbel_u)
